# baseline (speedup 1.0000x reference)
import functools

import jax
import jax.numpy as jnp
from jax import lax
from jax.experimental import pallas as pl
from jax.experimental.pallas import tpu as pltpu
from jax.experimental.pallas import tpu_sc as plsc

N = 16384
C = 1000
L = 16
NC, NS = 2, 16
NW = NC * NS
BPW = N // NW
WIN = 128
IPT = 32
NT = BPW // IPT

_LN2 = 0.6931471805599453
_P = (-1.9316664196629012, 3.4982118829630044, -2.4207929905996237,
      1.1047965807705125, -0.2806291682866353, 0.030102247599643327)


def _vlog(x):
    bits = lax.bitcast_convert_type(x, jnp.int32)
    e = lax.shift_right_logical(bits, 23) - 127
    m = lax.bitcast_convert_type((bits & 0x007FFFFF) | 0x3F800000, jnp.float32)
    p = _P[5]
    for c in (_P[4], _P[3], _P[2], _P[1], _P[0]):
        p = p * m + c
    return e.astype(jnp.float32) * _LN2 + p


_MESH = plsc.VectorSubcoreMesh(core_axis_name="c", subcore_axis_name="s")


@functools.partial(
    pl.kernel,
    mesh=_MESH,
    out_type=jax.ShapeDtypeStruct((NW, L), jnp.float32),
    compiler_params=pltpu.CompilerParams(needs_layout_passes=False,
                                         skip_device_barrier=True),
    scratch_types=[
        pltpu.VMEM((BPW,), jnp.int32),
        pltpu.VMEM((NT, IPT, WIN), jnp.float32),
        pltpu.VMEM((L,), jnp.float32),
        pltpu.SemaphoreType.DMA((NT,)),
    ],
)
def _nll_partials(tableT_hbm, tgt_hbm, out_hbm, tgt_v, win_v, acc_v, sem):
    wid = lax.axis_index("s") * NC + lax.axis_index("c")
    base = wid * BPW
    pltpu.sync_copy(tgt_hbm.at[pl.ds(base, BPW)], tgt_v)
    lane = lax.iota(jnp.int32, L)

    def _fire(w, carry):
        pltpu.async_copy(
            tableT_hbm.at[tgt_v.at[pl.ds(w * IPT, IPT)],
                          pl.ds(base + (w // 4) * WIN, WIN)],
            win_v.at[w], sem.at[w])
        return carry
    lax.fori_loop(0, NT, _fire, jnp.int32(0))

    def _window(w, a):
        pltpu.make_async_copy(
            tableT_hbm.at[tgt_v.at[pl.ds(w * IPT, IPT)],
                          pl.ds(base + (w // 4) * WIN, WIN)],
            win_v.at[w], sem.at[w]).wait()
        cbase = (w % 4) * IPT
        for j in range(IPT // L):
            d = j * L + lane
            a = a + _vlog(plsc.load_gather(win_v.at[w], [d, cbase + d]))
        return a
    acc = lax.fori_loop(0, NT, _window, jnp.zeros((L,), jnp.float32))

    acc_v[...] = acc * jnp.float32(-1.0 / N)
    pltpu.sync_copy(acc_v, out_hbm.at[wid])


def kernel(input, target):
    partials = _nll_partials(input.T, target.astype(jnp.int32))
    return jnp.sum(partials)
